# baseline (speedup 1.0000x reference)
import jax
import jax.numpy as jnp
from jax import lax
from jax.experimental import pallas as pl
from jax.experimental.pallas import tpu as pltpu
from jax.experimental.pallas import tpu_sc as plsc

_N = 10000
_E = 320000
_H = 128
_NC = 2
_NS = 16
_NW = _NC * _NS
_CH = 128
_NB = 2000
_EB = 2000

_f32 = jnp.float32


def _full(shape):
    return pl.BlockSpec(shape, lambda i: (0,) * len(shape))


def _rows(shape):
    return pl.BlockSpec(shape, lambda i: (i,) + (0,) * (len(shape) - 1))


def _relu(v):
    return jnp.maximum(v, 0.0)


def _enc_body(x_ref, a1, b1, a2, b2, out):
    t = _relu(x_ref[...] @ a1[...] + b1[...])
    out[...] = t @ a2[...] + b2[...]


def _edge_body(s1_ref, s2_ref, e_ref, w1, b1, w2, b2, m_out):
    msg_in = jnp.concatenate([s1_ref[...], s2_ref[...], e_ref[...]], axis=-1)
    t = _relu(msg_in @ w1[...] + b1[...])
    m_out[...] = t @ w2[...] + b2[...]


def _node_body(h_ref, a0_ref, a1_ref, a2_ref, a3_ref,
               u1, c1, u2, c2, h_out):
    h = h_ref[...]
    agg = (a0_ref[...] + a1_ref[...]) + (a2_ref[...] + a3_ref[...])
    nin = jnp.concatenate([h, agg], axis=-1)
    z = _relu(nin @ u1[...] + c1[...])
    h_out[...] = h + z @ u2[...] + c2[...]


def _node_final_body(h_ref, a0_ref, a1_ref, a2_ref, a3_ref,
                     u1, c1, u2, c2,
                     d1, g1, d2, g2, d3, g3, msk_ref, out_ref):
    h = h_ref[...]
    agg = (a0_ref[...] + a1_ref[...]) + (a2_ref[...] + a3_ref[...])
    nin = jnp.concatenate([h, agg], axis=-1)
    z = _relu(nin @ u1[...] + c1[...])
    hn = h + z @ u2[...] + c2[...]
    z1 = _relu(hn @ d1[...] + g1[...])
    z2 = _relu(z1 @ d2[...] + g2[...])
    out_ref[...] = (z2 @ d3[...] + g3[...]) * msk_ref[...]


def _nmat(n, d):
    return jax.ShapeDtypeStruct((n, d), _f32)


def _mlp2(xin, n, a1, b1, a2, b2, blk):
    return pl.pallas_call(
        _enc_body,
        grid=(n // blk,),
        in_specs=[_rows((blk, xin.shape[1])), _full(a1.shape), _full(b1.shape),
                  _full(a2.shape), _full(b2.shape)],
        out_specs=_rows((blk, _H)),
        out_shape=_nmat(n, _H),
    )(xin, a1, b1, a2, b2)


def _edge_mlp(s1, s2, e, w1, b1, w2, b2, h0, ne):
    eoff = h0 // _EB
    return pl.pallas_call(
        _edge_body,
        grid=(ne // _EB,),
        in_specs=[_rows((_EB, _H))] * 2 +
                 [pl.BlockSpec((_EB, _H), lambda i: (i + eoff, 0)),
                  _full(w1.shape), _full(b1.shape),
                  _full(w2.shape), _full(b2.shape)],
        out_specs=_rows((_EB, _H)),
        out_shape=_nmat(ne, _H),
    )(s1, s2, e, w1, b1, w2, b2)


def _node_update(h, a0, a1m, a2m, a3m, u1, c1, u2, c2):
    return pl.pallas_call(
        _node_body,
        grid=(_N // _NB,),
        in_specs=[_rows((_NB, _H))] * 5 +
                 [_full(w.shape) for w in (u1, c1, u2, c2)],
        out_specs=_rows((_NB, _H)),
        out_shape=_nmat(_N, _H),
    )(h, a0, a1m, a2m, a3m, u1, c1, u2, c2)


def _node_final(h, a0, a1m, a2m, a3m, u1, c1, u2, c2,
                d1, g1, d2, g2, d3, g3, msk):
    return pl.pallas_call(
        _node_final_body,
        grid=(_N // _NB,),
        in_specs=[_rows((_NB, _H))] * 5 +
                 [_full(w.shape) for w in (u1, c1, u2, c2,
                                           d1, g1, d2, g2, d3, g3)] +
                 [_rows((_NB, 3))],
        out_specs=_rows((_NB, 3)),
        out_shape=_nmat(_N, 3),
    )(h, a0, a1m, a2m, a3m, u1, c1, u2, c2,
      d1, g1, d2, g2, d3, g3, msk)


def _sc_mesh():
    return plsc.VectorSubcoreMesh(core_axis_name="c", subcore_axis_name="s",
                                  num_cores=_NC, num_subcores=_NS)


def _worker_id():
    return lax.axis_index("s") * _NC + lax.axis_index("c")


_GNB = 4


def _make_sc_gather(h0, ne):
    epw = ne // _NW
    gf = epw // _CH
    gt = epw - gf * _CH
    ntask = 2 * gf
    ngroups = ntask // _GNB
    leftover = ntask % _GNB

    def body(p_hbm, q_hbm, src_hbm, dst_hbm, s1_hbm, s2_hbm,
             sidx, didx, tidx, bufs, tbuf, *sems):
        sg = sems[:_GNB]
        sw = sems[_GNB:]
        w = _worker_id()
        e0 = h0 + w * epw
        o0 = w * epw

        pltpu.sync_copy(src_hbm.at[pl.ds(e0, gf * _CH)], sidx)
        pltpu.sync_copy(dst_hbm.at[pl.ds(e0, gf * _CH)], didx)

        def idx_of(j, b):
            ref = sidx if b % 2 == 0 else didx
            return ref.at[pl.ds(j * _CH, _CH)]

        def tab_of(b):
            return p_hbm if b % 2 == 0 else q_hbm

        def out_of(b):
            return s1_hbm if b % 2 == 0 else s2_hbm

        def start_gather(g, b):
            j = 2 * g + b // 2
            pltpu.async_copy(tab_of(b).at[idx_of(j, b)], bufs.at[b], sg[b])

        def wait_gather(g, b):
            j = 2 * g + b // 2
            pltpu.make_async_copy(tab_of(b).at[idx_of(j, b)], bufs.at[b],
                                  sg[b]).wait()

        def start_wb(g, b):
            j = 2 * g + b // 2
            pltpu.async_copy(bufs.at[b],
                             out_of(b).at[pl.ds(o0 + j * _CH, _CH)], sw[b])

        def wait_wb(g, b):
            j = 2 * g + b // 2
            pltpu.make_async_copy(bufs.at[b],
                                  out_of(b).at[pl.ds(o0 + j * _CH, _CH)],
                                  sw[b]).wait()

        def slot(g, b):
            pb = (b - 1) % _GNB
            pg = g if b > 0 else g - 1
            wait_gather(pg, pb)
            start_wb(pg, pb)
            wait_wb(g - 1, b)
            start_gather(g, b)

        start_gather(0, 0)
        for b in range(1, _GNB):
            wait_gather(0, b - 1)
            start_wb(0, b - 1)
            start_gather(0, b)

        def group(g, carry):
            for b in range(_GNB):
                slot(g, b)
            return carry

        lax.fori_loop(1, ngroups, group, 0)
        for b in range(leftover):
            slot(ngroups, b)

        tl = ntask - 1
        wait_gather(tl // _GNB, tl % _GNB)
        start_wb(tl // _GNB, tl % _GNB)
        for t in range(ntask - _GNB, ntask):
            wait_wb(t // _GNB, t % _GNB)

        if gt:
            pltpu.sync_copy(src_hbm.at[pl.ds(e0 + gf * _CH, gt)], tidx)
            pltpu.sync_copy(p_hbm.at[tidx], tbuf)
            pltpu.sync_copy(tbuf, s1_hbm.at[pl.ds(o0 + gf * _CH, gt)])
            pltpu.sync_copy(dst_hbm.at[pl.ds(e0 + gf * _CH, gt)], tidx)
            pltpu.sync_copy(q_hbm.at[tidx], tbuf)
            pltpu.sync_copy(tbuf, s2_hbm.at[pl.ds(o0 + gf * _CH, gt)])

    return pl.kernel(
        body,
        out_type=[_nmat(ne, _H), _nmat(ne, _H)],
        mesh=_sc_mesh(),
        scratch_types=[pltpu.VMEM((gf * _CH,), jnp.int32),
                       pltpu.VMEM((gf * _CH,), jnp.int32),
                       pltpu.VMEM((max(gt, 8),), jnp.int32),
                       pltpu.VMEM((_GNB, _CH, _H), _f32),
                       pltpu.VMEM((max(gt, 8), _H), _f32)] +
                      [pltpu.SemaphoreType.DMA] * (2 * _GNB),
    )


def _r1(b):
    return b.reshape(1, -1)


def kernel(x, coords, edge_attr, bc_disp, bc_rot, edge_index,
           enc_node, enc_edge, mp_params, dec):
    x2 = jnp.concatenate([coords, x[:, 3:]], axis=1)
    src = edge_index[0]
    dst = edge_index[1]
    mask3 = jnp.concatenate([1.0 - bc_disp, 1.0 - bc_disp, 1.0 - bc_rot],
                            axis=1)
    zeros_n = jnp.zeros((_N, _H), _f32)

    (ne1, nb1), (ne2, nb2) = enc_node
    (ee1, eb1), (ee2, eb2) = enc_edge

    e = _mlp2(edge_attr, _E, ee1, _r1(eb1), ee2, _r1(eb2), _EB)
    h = _mlp2(x2, _N, ne1, _r1(nb1), ne2, _r1(nb2), _NB)

    zl = zeros_n
    half = _E // 2
    gather_a = _make_sc_gather(0, half)
    gather_b = _make_sc_gather(half, half)

    pred = None
    for l in range(len(mp_params)):
        edge_mlp, node_mlp = mp_params[l]
        (w1, b1), (w2, b2) = edge_mlp
        (u1, c1), (u2, c2) = node_mlp

        s1a, s2a = gather_a(h, h, src, dst)
        s1b, s2b = gather_b(h, h, src, dst)
        ma = _edge_mlp(s1a, s2a, e, w1, _r1(b1), w2, _r1(b2), 0, half)
        mb = _edge_mlp(s1b, s2b, e, w1, _r1(b1), w2, _r1(b2), half, half)
        m = jnp.concatenate([ma, mb], axis=0)
        agg = jnp.zeros((_N, _H), _f32).at[dst].add(m)

        if l + 1 < len(mp_params):
            h = _node_update(h, agg, zl, zl, zl,
                             u1, _r1(c1), u2, _r1(c2))
        else:
            (d1, g1), (d2, g2), (d3, g3) = dec
            pred = _node_final(h, agg, zl, zl, zl,
                               u1, _r1(c1), u2, _r1(c2),
                               d1, _r1(g1), d2, _r1(g2), d3, _r1(g3), mask3)
    return pred


# speedup vs baseline: 1.0456x; 1.0456x over previous
import jax
import jax.numpy as jnp
from jax import lax
from jax.experimental import pallas as pl
from jax.experimental.pallas import tpu as pltpu
from jax.experimental.pallas import tpu_sc as plsc

_N = 10000
_E = 320000
_H = 128
_NC = 2
_NS = 16
_NW = _NC * _NS
_CH = 128
_NB = 2000
_EB = 2000

_f32 = jnp.float32


def _full(shape):
    return pl.BlockSpec(shape, lambda i: (0,) * len(shape))


def _rows(shape):
    return pl.BlockSpec(shape, lambda i: (i,) + (0,) * (len(shape) - 1))


def _relu(v):
    return jnp.maximum(v, 0.0)


def _enc_body(x_ref, a1, b1, a2, b2, out):
    t = _relu(x_ref[...] @ a1[...] + b1[...])
    out[...] = t @ a2[...] + b2[...]


def _edge_body(s1_ref, s2_ref, e_ref, w1, b1, w2, b2, m_out):
    msg_in = jnp.concatenate([s1_ref[...], s2_ref[...], e_ref[...]], axis=-1)
    t = _relu(msg_in @ w1[...] + b1[...])
    m_out[...] = t @ w2[...] + b2[...]


def _node_body(h_ref, a0_ref, a1_ref, a2_ref, a3_ref,
               u1, c1, u2, c2, h_out):
    h = h_ref[...]
    agg = (a0_ref[...] + a1_ref[...]) + (a2_ref[...] + a3_ref[...])
    nin = jnp.concatenate([h, agg], axis=-1)
    z = _relu(nin @ u1[...] + c1[...])
    h_out[...] = h + z @ u2[...] + c2[...]


def _node_final_body(h_ref, a0_ref, a1_ref, a2_ref, a3_ref,
                     u1, c1, u2, c2,
                     d1, g1, d2, g2, d3, g3, msk_ref, out_ref):
    h = h_ref[...]
    agg = (a0_ref[...] + a1_ref[...]) + (a2_ref[...] + a3_ref[...])
    nin = jnp.concatenate([h, agg], axis=-1)
    z = _relu(nin @ u1[...] + c1[...])
    hn = h + z @ u2[...] + c2[...]
    z1 = _relu(hn @ d1[...] + g1[...])
    z2 = _relu(z1 @ d2[...] + g2[...])
    out_ref[...] = (z2 @ d3[...] + g3[...]) * msk_ref[...]


def _nmat(n, d):
    return jax.ShapeDtypeStruct((n, d), _f32)


def _mlp2(xin, n, a1, b1, a2, b2, blk):
    return pl.pallas_call(
        _enc_body,
        grid=(n // blk,),
        in_specs=[_rows((blk, xin.shape[1])), _full(a1.shape), _full(b1.shape),
                  _full(a2.shape), _full(b2.shape)],
        out_specs=_rows((blk, _H)),
        out_shape=_nmat(n, _H),
    )(xin, a1, b1, a2, b2)


def _edge_mlp(s1, s2, e, w1, b1, w2, b2, h0, ne):
    eoff = h0 // _EB
    return pl.pallas_call(
        _edge_body,
        grid=(ne // _EB,),
        in_specs=[_rows((_EB, _H))] * 2 +
                 [pl.BlockSpec((_EB, _H), lambda i: (i + eoff, 0)),
                  _full(w1.shape), _full(b1.shape),
                  _full(w2.shape), _full(b2.shape)],
        out_specs=_rows((_EB, _H)),
        out_shape=_nmat(ne, _H),
    )(s1, s2, e, w1, b1, w2, b2)


def _node_update(h, a0, a1m, a2m, a3m, u1, c1, u2, c2):
    return pl.pallas_call(
        _node_body,
        grid=(_N // _NB,),
        in_specs=[_rows((_NB, _H))] * 5 +
                 [_full(w.shape) for w in (u1, c1, u2, c2)],
        out_specs=_rows((_NB, _H)),
        out_shape=_nmat(_N, _H),
    )(h, a0, a1m, a2m, a3m, u1, c1, u2, c2)


def _node_final(h, a0, a1m, a2m, a3m, u1, c1, u2, c2,
                d1, g1, d2, g2, d3, g3, msk):
    return pl.pallas_call(
        _node_final_body,
        grid=(_N // _NB,),
        in_specs=[_rows((_NB, _H))] * 5 +
                 [_full(w.shape) for w in (u1, c1, u2, c2,
                                           d1, g1, d2, g2, d3, g3)] +
                 [_rows((_NB, 3))],
        out_specs=_rows((_NB, 3)),
        out_shape=_nmat(_N, 3),
    )(h, a0, a1m, a2m, a3m, u1, c1, u2, c2,
      d1, g1, d2, g2, d3, g3, msk)


def _sc_mesh():
    return plsc.VectorSubcoreMesh(core_axis_name="c", subcore_axis_name="s",
                                  num_cores=_NC, num_subcores=_NS)


def _worker_id():
    return lax.axis_index("s") * _NC + lax.axis_index("c")


_GNB = 4


def _make_sc_gather(h0, ne):
    epw = ne // _NW
    gf = epw // _CH
    gt = epw - gf * _CH
    ntask = 2 * gf
    ngroups = ntask // _GNB
    leftover = ntask % _GNB

    def body(p_hbm, q_hbm, src_hbm, dst_hbm, s1_hbm, s2_hbm,
             sidx, didx, tidx, bufs, tbuf, *sems):
        sg = sems[:_GNB]
        sw = sems[_GNB:]
        w = _worker_id()
        e0 = h0 + w * epw
        o0 = w * epw

        pltpu.sync_copy(src_hbm.at[pl.ds(e0, gf * _CH)], sidx)
        pltpu.sync_copy(dst_hbm.at[pl.ds(e0, gf * _CH)], didx)

        def idx_of(j, b):
            ref = sidx if b % 2 == 0 else didx
            return ref.at[pl.ds(j * _CH, _CH)]

        def tab_of(b):
            return p_hbm if b % 2 == 0 else q_hbm

        def out_of(b):
            return s1_hbm if b % 2 == 0 else s2_hbm

        def start_gather(g, b):
            j = 2 * g + b // 2
            pltpu.async_copy(tab_of(b).at[idx_of(j, b)], bufs.at[b], sg[b])

        def wait_gather(g, b):
            j = 2 * g + b // 2
            pltpu.make_async_copy(tab_of(b).at[idx_of(j, b)], bufs.at[b],
                                  sg[b]).wait()

        def start_wb(g, b):
            j = 2 * g + b // 2
            pltpu.async_copy(bufs.at[b],
                             out_of(b).at[pl.ds(o0 + j * _CH, _CH)], sw[b])

        def wait_wb(g, b):
            j = 2 * g + b // 2
            pltpu.make_async_copy(bufs.at[b],
                                  out_of(b).at[pl.ds(o0 + j * _CH, _CH)],
                                  sw[b]).wait()

        def slot(g, b):
            pb = (b - 1) % _GNB
            pg = g if b > 0 else g - 1
            wait_gather(pg, pb)
            start_wb(pg, pb)
            wait_wb(g - 1, b)
            start_gather(g, b)

        start_gather(0, 0)
        for b in range(1, _GNB):
            wait_gather(0, b - 1)
            start_wb(0, b - 1)
            start_gather(0, b)

        def group(g, carry):
            for b in range(_GNB):
                slot(g, b)
            return carry

        lax.fori_loop(1, ngroups, group, 0)
        for b in range(leftover):
            slot(ngroups, b)

        tl = ntask - 1
        wait_gather(tl // _GNB, tl % _GNB)
        start_wb(tl // _GNB, tl % _GNB)
        for t in range(ntask - _GNB, ntask):
            wait_wb(t // _GNB, t % _GNB)

        if gt:
            pltpu.sync_copy(src_hbm.at[pl.ds(e0 + gf * _CH, gt)], tidx)
            pltpu.sync_copy(p_hbm.at[tidx], tbuf)
            pltpu.sync_copy(tbuf, s1_hbm.at[pl.ds(o0 + gf * _CH, gt)])
            pltpu.sync_copy(dst_hbm.at[pl.ds(e0 + gf * _CH, gt)], tidx)
            pltpu.sync_copy(q_hbm.at[tidx], tbuf)
            pltpu.sync_copy(tbuf, s2_hbm.at[pl.ds(o0 + gf * _CH, gt)])

    return pl.kernel(
        body,
        out_type=[_nmat(ne, _H), _nmat(ne, _H)],
        mesh=_sc_mesh(),
        scratch_types=[pltpu.VMEM((gf * _CH,), jnp.int32),
                       pltpu.VMEM((gf * _CH,), jnp.int32),
                       pltpu.VMEM((max(gt, 8),), jnp.int32),
                       pltpu.VMEM((_GNB, _CH, _H), _f32),
                       pltpu.VMEM((max(gt, 8), _H), _f32)] +
                      [pltpu.SemaphoreType.DMA] * (2 * _GNB),
    )


def _r1(b):
    return b.reshape(1, -1)


def kernel(x, coords, edge_attr, bc_disp, bc_rot, edge_index,
           enc_node, enc_edge, mp_params, dec):
    x2 = jnp.concatenate([coords, x[:, 3:]], axis=1)
    src = edge_index[0]
    dst = edge_index[1]
    mask3 = jnp.concatenate([1.0 - bc_disp, 1.0 - bc_disp, 1.0 - bc_rot],
                            axis=1)
    zeros_n = jnp.zeros((_N, _H), _f32)

    (ne1, nb1), (ne2, nb2) = enc_node
    (ee1, eb1), (ee2, eb2) = enc_edge

    e = _mlp2(edge_attr, _E, ee1, _r1(eb1), ee2, _r1(eb2), _EB)
    h = _mlp2(x2, _N, ne1, _r1(nb1), ne2, _r1(nb2), _NB)

    zl = zeros_n
    gather_all = _make_sc_gather(0, _E)

    pred = None
    for l in range(len(mp_params)):
        edge_mlp, node_mlp = mp_params[l]
        (w1, b1), (w2, b2) = edge_mlp
        (u1, c1), (u2, c2) = node_mlp

        s1, s2 = gather_all(h, h, src, dst)
        m = _edge_mlp(s1, s2, e, w1, _r1(b1), w2, _r1(b2), 0, _E)
        agg = jnp.zeros((_N, _H), _f32).at[dst].add(m)

        if l + 1 < len(mp_params):
            h = _node_update(h, agg, zl, zl, zl,
                             u1, _r1(c1), u2, _r1(c2))
        else:
            (d1, g1), (d2, g2), (d3, g3) = dec
            pred = _node_final(h, agg, zl, zl, zl,
                               u1, _r1(c1), u2, _r1(c2),
                               d1, _r1(g1), d2, _r1(g2), d3, _r1(g3), mask3)
    return pred


# speedup vs baseline: 1.0495x; 1.0038x over previous
import jax
import jax.numpy as jnp
from jax import lax
from jax.experimental import pallas as pl
from jax.experimental.pallas import tpu as pltpu
from jax.experimental.pallas import tpu_sc as plsc

_N = 10000
_E = 320000
_H = 128
_NC = 2
_NS = 16
_NW = _NC * _NS
_CH = 128
_NB = 2000
_EB = 2000

_f32 = jnp.float32


def _full(shape):
    return pl.BlockSpec(shape, lambda i: (0,) * len(shape))


def _rows(shape):
    return pl.BlockSpec(shape, lambda i: (i,) + (0,) * (len(shape) - 1))


def _relu(v):
    return jnp.maximum(v, 0.0)


def _enc_body(x_ref, a1, b1, a2, b2, out):
    t = _relu(x_ref[...] @ a1[...] + b1[...])
    out[...] = t @ a2[...] + b2[...]


def _edge_body(s1_ref, s2_ref, e_ref, w1, b1, w2, b2, m_out):
    msg_in = jnp.concatenate([s1_ref[...], s2_ref[...], e_ref[...]], axis=-1)
    t = _relu(msg_in @ w1[...] + b1[...])
    m_out[...] = t @ w2[...] + b2[...]


def _node_body(h_ref, a0_ref, u1, c1, u2, c2, h_out):
    h = h_ref[...]
    agg = a0_ref[...]
    nin = jnp.concatenate([h, agg], axis=-1)
    z = _relu(nin @ u1[...] + c1[...])
    h_out[...] = h + z @ u2[...] + c2[...]


def _node_final_body(h_ref, a0_ref, u1, c1, u2, c2,
                     d1, g1, d2, g2, d3, g3, msk_ref, out_ref):
    h = h_ref[...]
    agg = a0_ref[...]
    nin = jnp.concatenate([h, agg], axis=-1)
    z = _relu(nin @ u1[...] + c1[...])
    hn = h + z @ u2[...] + c2[...]
    z1 = _relu(hn @ d1[...] + g1[...])
    z2 = _relu(z1 @ d2[...] + g2[...])
    out_ref[...] = (z2 @ d3[...] + g3[...]) * msk_ref[...]


def _nmat(n, d):
    return jax.ShapeDtypeStruct((n, d), _f32)


def _mlp2(xin, n, a1, b1, a2, b2, blk):
    return pl.pallas_call(
        _enc_body,
        grid=(n // blk,),
        in_specs=[_rows((blk, xin.shape[1])), _full(a1.shape), _full(b1.shape),
                  _full(a2.shape), _full(b2.shape)],
        out_specs=_rows((blk, _H)),
        out_shape=_nmat(n, _H),
    )(xin, a1, b1, a2, b2)


def _edge_mlp(s1, s2, e, w1, b1, w2, b2, h0, ne):
    eoff = h0 // _EB
    return pl.pallas_call(
        _edge_body,
        grid=(ne // _EB,),
        in_specs=[_rows((_EB, _H))] * 2 +
                 [pl.BlockSpec((_EB, _H), lambda i: (i + eoff, 0)),
                  _full(w1.shape), _full(b1.shape),
                  _full(w2.shape), _full(b2.shape)],
        out_specs=_rows((_EB, _H)),
        out_shape=_nmat(ne, _H),
    )(s1, s2, e, w1, b1, w2, b2)


def _node_update(h, a0, u1, c1, u2, c2):
    return pl.pallas_call(
        _node_body,
        grid=(_N // _NB,),
        in_specs=[_rows((_NB, _H))] * 2 +
                 [_full(w.shape) for w in (u1, c1, u2, c2)],
        out_specs=_rows((_NB, _H)),
        out_shape=_nmat(_N, _H),
    )(h, a0, u1, c1, u2, c2)


def _node_final(h, a0, u1, c1, u2, c2,
                d1, g1, d2, g2, d3, g3, msk):
    return pl.pallas_call(
        _node_final_body,
        grid=(_N // _NB,),
        in_specs=[_rows((_NB, _H))] * 2 +
                 [_full(w.shape) for w in (u1, c1, u2, c2,
                                           d1, g1, d2, g2, d3, g3)] +
                 [_rows((_NB, 3))],
        out_specs=_rows((_NB, 3)),
        out_shape=_nmat(_N, 3),
    )(h, a0, u1, c1, u2, c2,
      d1, g1, d2, g2, d3, g3, msk)


def _sc_mesh():
    return plsc.VectorSubcoreMesh(core_axis_name="c", subcore_axis_name="s",
                                  num_cores=_NC, num_subcores=_NS)


def _worker_id():
    return lax.axis_index("s") * _NC + lax.axis_index("c")


_GNB = 4


def _make_sc_gather(h0, ne):
    epw = ne // _NW
    gf = epw // _CH
    gt = epw - gf * _CH
    ntask = 2 * gf
    ngroups = ntask // _GNB
    leftover = ntask % _GNB

    def body(p_hbm, q_hbm, src_hbm, dst_hbm, s1_hbm, s2_hbm,
             sidx, didx, tidx, bufs, tbuf, *sems):
        sg = sems[:_GNB]
        sw = sems[_GNB:]
        w = _worker_id()
        e0 = h0 + w * epw
        o0 = w * epw

        pltpu.sync_copy(src_hbm.at[pl.ds(e0, gf * _CH)], sidx)
        pltpu.sync_copy(dst_hbm.at[pl.ds(e0, gf * _CH)], didx)

        def idx_of(j, b):
            ref = sidx if b % 2 == 0 else didx
            return ref.at[pl.ds(j * _CH, _CH)]

        def tab_of(b):
            return p_hbm if b % 2 == 0 else q_hbm

        def out_of(b):
            return s1_hbm if b % 2 == 0 else s2_hbm

        def start_gather(g, b):
            j = 2 * g + b // 2
            pltpu.async_copy(tab_of(b).at[idx_of(j, b)], bufs.at[b], sg[b])

        def wait_gather(g, b):
            j = 2 * g + b // 2
            pltpu.make_async_copy(tab_of(b).at[idx_of(j, b)], bufs.at[b],
                                  sg[b]).wait()

        def start_wb(g, b):
            j = 2 * g + b // 2
            pltpu.async_copy(bufs.at[b],
                             out_of(b).at[pl.ds(o0 + j * _CH, _CH)], sw[b])

        def wait_wb(g, b):
            j = 2 * g + b // 2
            pltpu.make_async_copy(bufs.at[b],
                                  out_of(b).at[pl.ds(o0 + j * _CH, _CH)],
                                  sw[b]).wait()

        def slot(g, b):
            pb = (b - 1) % _GNB
            pg = g if b > 0 else g - 1
            wait_gather(pg, pb)
            start_wb(pg, pb)
            wait_wb(g - 1, b)
            start_gather(g, b)

        start_gather(0, 0)
        for b in range(1, _GNB):
            wait_gather(0, b - 1)
            start_wb(0, b - 1)
            start_gather(0, b)

        def group(g, carry):
            for b in range(_GNB):
                slot(g, b)
            return carry

        lax.fori_loop(1, ngroups, group, 0)
        for b in range(leftover):
            slot(ngroups, b)

        tl = ntask - 1
        wait_gather(tl // _GNB, tl % _GNB)
        start_wb(tl // _GNB, tl % _GNB)
        for t in range(ntask - _GNB, ntask):
            wait_wb(t // _GNB, t % _GNB)

        if gt:
            pltpu.sync_copy(src_hbm.at[pl.ds(e0 + gf * _CH, gt)], tidx)
            pltpu.sync_copy(p_hbm.at[tidx], tbuf)
            pltpu.sync_copy(tbuf, s1_hbm.at[pl.ds(o0 + gf * _CH, gt)])
            pltpu.sync_copy(dst_hbm.at[pl.ds(e0 + gf * _CH, gt)], tidx)
            pltpu.sync_copy(q_hbm.at[tidx], tbuf)
            pltpu.sync_copy(tbuf, s2_hbm.at[pl.ds(o0 + gf * _CH, gt)])

    return pl.kernel(
        body,
        out_type=[_nmat(ne, _H), _nmat(ne, _H)],
        mesh=_sc_mesh(),
        scratch_types=[pltpu.VMEM((gf * _CH,), jnp.int32),
                       pltpu.VMEM((gf * _CH,), jnp.int32),
                       pltpu.VMEM((max(gt, 8),), jnp.int32),
                       pltpu.VMEM((_GNB, _CH, _H), _f32),
                       pltpu.VMEM((max(gt, 8), _H), _f32)] +
                      [pltpu.SemaphoreType.DMA] * (2 * _GNB),
    )


def _r1(b):
    return b.reshape(1, -1)


def kernel(x, coords, edge_attr, bc_disp, bc_rot, edge_index,
           enc_node, enc_edge, mp_params, dec):
    x2 = jnp.concatenate([coords, x[:, 3:]], axis=1)
    src = edge_index[0]
    dst = edge_index[1]
    mask3 = jnp.concatenate([1.0 - bc_disp, 1.0 - bc_disp, 1.0 - bc_rot],
                            axis=1)

    (ne1, nb1), (ne2, nb2) = enc_node
    (ee1, eb1), (ee2, eb2) = enc_edge

    e = _mlp2(edge_attr, _E, ee1, _r1(eb1), ee2, _r1(eb2), _EB)
    h = _mlp2(x2, _N, ne1, _r1(nb1), ne2, _r1(nb2), _NB)

    gather_all = _make_sc_gather(0, _E)

    pred = None
    for l in range(len(mp_params)):
        edge_mlp, node_mlp = mp_params[l]
        (w1, b1), (w2, b2) = edge_mlp
        (u1, c1), (u2, c2) = node_mlp

        s1, s2 = gather_all(h, h, src, dst)
        m = _edge_mlp(s1, s2, e, w1, _r1(b1), w2, _r1(b2), 0, _E)
        agg = jnp.zeros((_N, _H), _f32).at[dst].add(m)

        if l + 1 < len(mp_params):
            h = _node_update(h, agg, u1, _r1(c1), u2, _r1(c2))
        else:
            (d1, g1), (d2, g2), (d3, g3) = dec
            pred = _node_final(h, agg, u1, _r1(c1), u2, _r1(c2),
                               d1, _r1(g1), d2, _r1(g2), d3, _r1(g3), mask3)
    return pred
